# baseline (device time: 147427 ns/iter reference)
import jax
import jax.numpy as jnp
from jax import lax
from jax.experimental import pallas as pl
from jax.experimental.pallas import tpu as pltpu


def kernel(Q, K, V):
    b, sq, h, d = Q.shape
    bh = b * h
    scale = d ** -0.5

    Qt = jnp.transpose(Q, (0, 2, 1, 3)).reshape(bh, sq, d).astype(jnp.bfloat16)
    Kt = jnp.transpose(K, (0, 2, 1, 3)).reshape(bh, sq, d).astype(jnp.bfloat16)
    Vt = jnp.transpose(V, (0, 2, 1, 3)).reshape(bh, sq, d).astype(jnp.bfloat16)

    def body(q_ref, k_ref, v_ref, o_ref, kr_ref, vr_ref, send_sems, recv_sems):
        my_x = lax.axis_index("x")
        my_y = lax.axis_index("y")
        nbr = (1 - my_x, my_y)

        barrier = pltpu.get_barrier_semaphore()
        pl.semaphore_signal(
            barrier, inc=1, device_id=nbr, device_id_type=pl.DeviceIdType.MESH
        )
        pl.semaphore_wait(barrier, 1)

        rk = pltpu.make_async_remote_copy(
            src_ref=k_ref, dst_ref=kr_ref,
            send_sem=send_sems.at[0], recv_sem=recv_sems.at[0],
            device_id=nbr, device_id_type=pl.DeviceIdType.MESH,
        )
        rv = pltpu.make_async_remote_copy(
            src_ref=v_ref, dst_ref=vr_ref,
            send_sem=send_sems.at[1], recv_sem=recv_sems.at[1],
            device_id=nbr, device_id_type=pl.DeviceIdType.MESH,
        )
        rk.start()
        rv.start()
        rk.wait()
        rv.wait()

        def step(i, _):
            q = q_ref[i]
            k_all = jnp.concatenate([k_ref[i], kr_ref[i]], axis=0)
            v_all = jnp.concatenate([v_ref[i], vr_ref[i]], axis=0)
            s = lax.dot_general(
                q, k_all, (((1,), (1,)), ((), ())),
                preferred_element_type=jnp.float32,
            ) * scale
            m = jnp.max(s, axis=-1, keepdims=True)
            p = jnp.exp(s - m)
            l = jnp.sum(p, axis=-1, keepdims=True)
            p = (p / l).astype(jnp.bfloat16)
            o_ref[i] = lax.dot_general(
                p, v_all, (((1,), (0,)), ((), ())),
                preferred_element_type=jnp.float32,
            )
            return 0

        lax.fori_loop(0, bh, step, 0)

    out = pl.pallas_call(
        body,
        out_shape=jax.ShapeDtypeStruct((bh, sq, d), jnp.float32),
        in_specs=[pl.BlockSpec(memory_space=pltpu.VMEM)] * 3,
        out_specs=pl.BlockSpec(memory_space=pltpu.VMEM),
        scratch_shapes=[
            pltpu.VMEM((bh, sq, d), jnp.bfloat16),
            pltpu.VMEM((bh, sq, d), jnp.bfloat16),
            pltpu.SemaphoreType.DMA((2,)),
            pltpu.SemaphoreType.DMA((2,)),
        ],
        compiler_params=pltpu.CompilerParams(collective_id=0),
    )(Qt, Kt, Vt)

    return jnp.transpose(out.reshape(b, h, sq, d), (0, 2, 1, 3))
